# Initial kernel scaffold; baseline (speedup 1.0000x reference)
import jax
import jax.numpy as jnp
from jax.experimental import pallas as pl


def kernel(x, pos_emb):
    raise NotImplementedError("write your pallas kernel here")


# speedup vs baseline: 3.8659x; 3.8659x over previous
import jax
import jax.numpy as jnp
from jax.experimental import pallas as pl
from jax.experimental.pallas import tpu as pltpu

S_BLK = 512


def _add_pe_kernel(x_ref, pe_ref, o_ref):
    pe = pe_ref[...]
    o_ref[...] = x_ref[...] + pe[:, None, :]


def kernel(x, pos_emb):
    seq_len, batch, d_model = x.shape
    grid = (seq_len // S_BLK,)
    return pl.pallas_call(
        _add_pe_kernel,
        grid=grid,
        in_specs=[
            pl.BlockSpec((S_BLK, batch, d_model), lambda i: (i, 0, 0)),
            pl.BlockSpec((S_BLK, d_model), lambda i: (i, 0)),
        ],
        out_specs=pl.BlockSpec((S_BLK, batch, d_model), lambda i: (i, 0, 0)),
        out_shape=jax.ShapeDtypeStruct((seq_len, batch, d_model), x.dtype),
        compiler_params=pltpu.CompilerParams(
            dimension_semantics=("arbitrary",),
        ),
    )(x, pos_emb[:seq_len])
